# baseline (speedup 1.0000x reference)
import functools

import jax
import jax.numpy as jnp
from jax import lax
from jax.experimental import pallas as pl
from jax.experimental.pallas import tpu as pltpu
from jax.experimental.pallas import tpu_sc as plsc

_VOCAB = 50
_HIDDEN = 32
_BATCH = 4096
_SEQ = 200
_N = _BATCH * _SEQ

_TPAD = 64
_NC = 2
_NS = 16
_NW = _NC * _NS
_B_PER_W = _BATCH // _NW
_NB = 4
_TOK = _NB * _SEQ
_GROUPS = _TOK // 16
_N_CHUNKS = _B_PER_W // _NB


def _fuse_table_body(e_ref, w_ref, b_ref, t_ref):
    t_ref[...] = (
        jnp.dot(e_ref[...], w_ref[...], preferred_element_type=jnp.float32)
        + b_ref[...]
    )


def _fuse_table(embed_table, W, b):
    wp = jnp.zeros((_HIDDEN, _TPAD), jnp.float32).at[:, :_VOCAB].set(W)
    bp = jnp.zeros((1, _TPAD), jnp.float32).at[0, :_VOCAB].set(b)
    t = pl.pallas_call(
        _fuse_table_body,
        out_shape=jax.ShapeDtypeStruct((_VOCAB, _TPAD), jnp.float32),
    )(embed_table, wp, bp)
    return t.reshape(_VOCAB * _TPAD)


def _gather_body(t_hbm, idx_hbm, out_hbm, table_v, idx_v, out_v, sem):
    wid = lax.axis_index("s") * _NC + lax.axis_index("c")
    base_b = wid * _B_PER_W

    pltpu.sync_copy(t_hbm, table_v)

    lane = lax.iota(jnp.int32, 16)

    def chunk_body(c, _):
        b0 = base_b + c * _NB
        pltpu.sync_copy(idx_hbm.at[pl.ds(b0 * _SEQ, _TOK)], idx_v)

        def group_body(g, _):
            tt = g * 16 + lane
            jb = tt // _SEQ
            s = tt - jb * _SEQ
            rowids = idx_v[pl.ds(g * 16, 16)]
            fb = rowids * _TPAD
            for col in range(_VOCAB):
                vals = plsc.load_gather(table_v, [fb + col])
                colv = jnp.full((16,), col, jnp.int32)
                plsc.store_scatter(out_v, [jb, s, colv], vals)
            return 0

        lax.fori_loop(0, _GROUPS, group_body, 0)
        pltpu.sync_copy(out_v, out_hbm.at[pl.ds(b0, _NB)])
        return 0

    lax.fori_loop(0, _N_CHUNKS, chunk_body, 0)


def _sc_gather(table_flat, idx_flat):
    mesh = plsc.VectorSubcoreMesh(core_axis_name="c", subcore_axis_name="s")
    kern = functools.partial(
        pl.kernel,
        mesh=mesh,
        compiler_params=pltpu.CompilerParams(needs_layout_passes=False),
        out_type=jax.ShapeDtypeStruct((_BATCH, _SEQ, _VOCAB), jnp.float32),
        scratch_types=[
            pltpu.VMEM((_VOCAB * _TPAD,), jnp.float32),
            pltpu.VMEM((_TOK,), jnp.int32),
            pltpu.VMEM((_NB, _SEQ, _VOCAB), jnp.float32),
            pltpu.SemaphoreType.DMA,
        ],
    )(_gather_body)
    return kern(table_flat, idx_flat)


def kernel(indices, embed_table, W, b):
    table_flat = _fuse_table(embed_table, W, b)
    return _sc_gather(table_flat, indices.reshape(_N))


# speedup vs baseline: 1.1803x; 1.1803x over previous
import functools

import jax
import jax.numpy as jnp
from jax import lax
from jax.experimental import pallas as pl
from jax.experimental.pallas import tpu as pltpu
from jax.experimental.pallas import tpu_sc as plsc

_VOCAB = 50
_HIDDEN = 32
_BATCH = 4096
_SEQ = 200
_N = _BATCH * _SEQ

_TPAD = 64
_NC = 2
_NS = 16
_NW = _NC * _NS
_B_PER_W = _BATCH // _NW
_NB = 4
_TOK = _NB * _SEQ
_GROUPS = _TOK // 16
_N_CHUNKS = _B_PER_W // _NB


def _fuse_table_body(e_ref, w_ref, b_ref, t_ref):
    t_ref[...] = (
        jnp.dot(e_ref[...], w_ref[...], preferred_element_type=jnp.float32)
        + b_ref[...]
    )


def _fuse_table(embed_table, W, b):
    wp = jnp.zeros((_HIDDEN, _TPAD), jnp.float32).at[:, :_VOCAB].set(W)
    bp = jnp.zeros((1, _TPAD), jnp.float32).at[0, :_VOCAB].set(b)
    t = pl.pallas_call(
        _fuse_table_body,
        out_shape=jax.ShapeDtypeStruct((_VOCAB, _TPAD), jnp.float32),
    )(embed_table, wp, bp)
    return t.reshape(_VOCAB * _TPAD)


def _gather_body(t_hbm, idx_hbm, out_hbm, table_v, idx_v, out_v, sem):
    wid = lax.axis_index("s") * _NC + lax.axis_index("c")
    base_b = wid * _B_PER_W

    pltpu.sync_copy(t_hbm, table_v)

    lane = lax.iota(jnp.int32, 16)

    def chunk_body(c, _):
        b0 = base_b + c * _NB
        pltpu.sync_copy(idx_hbm.at[pl.ds(b0 * _SEQ, _TOK)], idx_v)

        def group_body(g, _):
            tt = g * 16 + lane
            jb = tt // _SEQ
            s = tt - jb * _SEQ
            rowids = idx_v[pl.ds(g * 16, 16)]
            fb = rowids * _TPAD
            for col in range(_VOCAB):
                vals = plsc.load_gather(table_v, [fb + col])
                colv = jnp.full((16,), col, jnp.int32)
                plsc.store_scatter(out_v, [jb, s, colv], vals)
            return 0

        lax.fori_loop(0, _GROUPS, group_body, 0)
        pltpu.sync_copy(out_v, out_hbm.at[pl.ds(b0, _NB)])
        return 0

    lax.fori_loop(0, _N_CHUNKS, chunk_body, 0)


def _sc_gather(table_flat, idx_flat):
    mesh = plsc.VectorSubcoreMesh(core_axis_name="c", subcore_axis_name="s")
    kern = functools.partial(
        pl.kernel,
        mesh=mesh,
        compiler_params=pltpu.CompilerParams(
            needs_layout_passes=False, use_tc_tiling_on_sc=False
        ),
        out_type=jax.ShapeDtypeStruct((_BATCH, _SEQ, _VOCAB), jnp.float32),
        scratch_types=[
            pltpu.VMEM((_VOCAB * _TPAD,), jnp.float32),
            pltpu.VMEM((_TOK,), jnp.int32),
            pltpu.VMEM((_NB, _SEQ, _VOCAB), jnp.float32),
            pltpu.SemaphoreType.DMA,
        ],
    )(_gather_body)
    return kern(table_flat, idx_flat)


def kernel(indices, embed_table, W, b):
    table_flat = _fuse_table(embed_table, W, b)
    return _sc_gather(table_flat, indices.reshape(_N))


# speedup vs baseline: 1.2367x; 1.0478x over previous
import functools

import jax
import jax.numpy as jnp
from jax import lax
from jax.experimental import pallas as pl
from jax.experimental.pallas import tpu as pltpu
from jax.experimental.pallas import tpu_sc as plsc

_VOCAB = 50
_HIDDEN = 32
_BATCH = 4096
_SEQ = 200
_N = _BATCH * _SEQ

_TPAD = 64
_NC = 2
_NS = 16
_NW = _NC * _NS
_B_PER_W = _BATCH // _NW
_NB = 4
_TOK = _NB * _SEQ
_GROUPS = _TOK // 16
_N_CHUNKS = _B_PER_W // _NB


def _fuse_table_body(e_ref, w_ref, b_ref, t_ref):
    t_ref[...] = (
        jnp.dot(e_ref[...], w_ref[...], preferred_element_type=jnp.float32)
        + b_ref[...]
    )


def _fuse_table(embed_table, W, b):
    wp = jnp.zeros((_HIDDEN, _TPAD), jnp.float32).at[:, :_VOCAB].set(W)
    bp = jnp.zeros((1, _TPAD), jnp.float32).at[0, :_VOCAB].set(b)
    t = pl.pallas_call(
        _fuse_table_body,
        out_shape=jax.ShapeDtypeStruct((_VOCAB, _TPAD), jnp.float32),
    )(embed_table, wp, bp)
    return t.reshape(_VOCAB * _TPAD)


def _gather_body(t_hbm, idx_hbm, out_hbm,
                 table_v, idx_v0, idx_v1, out_v0, out_v1,
                 sem_t, sem_i0, sem_i1, sem_o0, sem_o1):
    wid = lax.axis_index("s") * _NC + lax.axis_index("c")
    base_b = wid * _B_PER_W

    pltpu.sync_copy(t_hbm, table_v)

    lane = lax.iota(jnp.int32, 16)
    idx_bufs = (idx_v0, idx_v1)
    out_bufs = (out_v0, out_v1)
    idx_sems = (sem_i0, sem_i1)
    out_sems = (sem_o0, sem_o1)

    def start_idx(c, buf, sem):
        b0 = base_b + c * _NB
        pltpu.async_copy(idx_hbm.at[pl.ds(b0 * _SEQ, _TOK)], buf, sem)

    def compute(idx_v, out_v):
        def group_body(g, _):
            tt = g * 16 + lane
            jb = tt // _SEQ
            s = tt - jb * _SEQ
            rowids = idx_v[pl.ds(g * 16, 16)]
            fb = rowids * _TPAD
            for col in range(_VOCAB):
                vals = plsc.load_gather(table_v, [fb + col])
                colv = jnp.full((16,), col, jnp.int32)
                plsc.store_scatter(out_v, [jb, s, colv], vals)
            return 0

        lax.fori_loop(0, _GROUPS, group_body, 0)

    start_idx(0, idx_bufs[0], idx_sems[0])

    def pair_body(cc, _):
        for bsel in range(2):
            c = cc * 2 + bsel
            b0 = base_b + c * _NB
            ivb, isem = idx_bufs[bsel], idx_sems[bsel]
            ovb, osem = out_bufs[bsel], out_sems[bsel]
            pltpu.make_async_copy(
                idx_hbm.at[pl.ds(b0 * _SEQ, _TOK)], ivb, isem
            ).wait()

            @pl.when(c + 1 < _N_CHUNKS)
            def _():
                b1 = base_b + (c + 1) * _NB
                pltpu.async_copy(
                    idx_hbm.at[pl.ds(b1 * _SEQ, _TOK)],
                    idx_bufs[1 - bsel], idx_sems[1 - bsel],
                )

            @pl.when(c >= 2)
            def _():
                pltpu.make_async_copy(
                    ovb, out_hbm.at[pl.ds(b0, _NB)], osem
                ).wait()

            compute(ivb, ovb)
            pltpu.async_copy(ovb, out_hbm.at[pl.ds(b0, _NB)], osem)
        return 0

    lax.fori_loop(0, _N_CHUNKS // 2, pair_body, 0)

    for bsel in range(2):
        c = _N_CHUNKS - 2 + bsel
        b0 = base_b + c * _NB
        pltpu.make_async_copy(
            out_bufs[bsel], out_hbm.at[pl.ds(b0, _NB)], out_sems[bsel]
        ).wait()


def _sc_gather(table_flat, idx_flat):
    mesh = plsc.VectorSubcoreMesh(core_axis_name="c", subcore_axis_name="s")
    kern = functools.partial(
        pl.kernel,
        mesh=mesh,
        compiler_params=pltpu.CompilerParams(
            needs_layout_passes=False, use_tc_tiling_on_sc=False
        ),
        out_type=jax.ShapeDtypeStruct((_BATCH, _SEQ, _VOCAB), jnp.float32),
        scratch_types=[
            pltpu.VMEM((_VOCAB * _TPAD,), jnp.float32),
            pltpu.VMEM((_TOK,), jnp.int32),
            pltpu.VMEM((_TOK,), jnp.int32),
            pltpu.VMEM((_NB, _SEQ, _VOCAB), jnp.float32),
            pltpu.VMEM((_NB, _SEQ, _VOCAB), jnp.float32),
            pltpu.SemaphoreType.DMA,
            pltpu.SemaphoreType.DMA,
            pltpu.SemaphoreType.DMA,
            pltpu.SemaphoreType.DMA,
            pltpu.SemaphoreType.DMA,
        ],
    )(_gather_body)
    return kern(table_flat, idx_flat)


def kernel(indices, embed_table, W, b):
    table_flat = _fuse_table(embed_table, W, b)
    return _sc_gather(table_flat, indices.reshape(_N))


# speedup vs baseline: 2.0565x; 1.6628x over previous
import functools

import jax
import jax.numpy as jnp
from jax import lax
from jax.experimental import pallas as pl
from jax.experimental.pallas import tpu as pltpu
from jax.experimental.pallas import tpu_sc as plsc

_VOCAB = 50
_HIDDEN = 32
_BATCH = 4096
_SEQ = 200
_N = _BATCH * _SEQ

_TPAD = 65
_NC = 2
_NS = 16
_NW = _NC * _NS
_B_PER_W = _BATCH // _NW
_NB = 4
_TOK = _NB * _SEQ
_GROUPS = _TOK // 16
_N_CHUNKS = _B_PER_W // _NB


def _fuse_table_body(e_ref, w_ref, b_ref, t_ref):
    t_ref[...] = (
        jnp.dot(e_ref[...], w_ref[...], preferred_element_type=jnp.float32)
        + b_ref[...]
    )


def _fuse_table(embed_table, W, b):
    wp = jnp.zeros((_HIDDEN, _TPAD), jnp.float32).at[:, :_VOCAB].set(W)
    bp = jnp.zeros((1, _TPAD), jnp.float32).at[0, :_VOCAB].set(b)
    t = pl.pallas_call(
        _fuse_table_body,
        out_shape=jax.ShapeDtypeStruct((_VOCAB, _TPAD), jnp.float32),
    )(embed_table, wp, bp)
    return t.reshape(_VOCAB * _TPAD)


def _gather_body(t_hbm, idx_hbm, out_hbm,
                 table_v, idx_v0, idx_v1, out_v0, out_v1,
                 sem_t, sem_i0, sem_i1, sem_o0, sem_o1):
    wid = lax.axis_index("s") * _NC + lax.axis_index("c")
    base_b = wid * _B_PER_W

    pltpu.sync_copy(t_hbm, table_v)

    lane = lax.iota(jnp.int32, 16)
    idx_bufs = (idx_v0, idx_v1)
    out_bufs = (out_v0, out_v1)
    idx_sems = (sem_i0, sem_i1)
    out_sems = (sem_o0, sem_o1)

    def start_idx(c, buf, sem):
        b0 = base_b + c * _NB
        pltpu.async_copy(idx_hbm.at[pl.ds(b0 * _SEQ, _TOK)], buf, sem)

    def compute(idx_v, out_v):
        def group_body(g, _):
            tt = g * 16 + lane
            jb = tt // _SEQ
            s = tt - jb * _SEQ
            rowids = idx_v[pl.ds(g * 16, 16)]
            fb = rowids * _TPAD
            for col in range(_VOCAB):
                vals = plsc.load_gather(table_v, [fb + col])
                colv = jnp.full((16,), col, jnp.int32)
                plsc.store_scatter(out_v, [jb, s, colv], vals)
            return 0

        lax.fori_loop(0, _GROUPS, group_body, 0)

    start_idx(0, idx_bufs[0], idx_sems[0])

    def pair_body(cc, _):
        for bsel in range(2):
            c = cc * 2 + bsel
            b0 = base_b + c * _NB
            ivb, isem = idx_bufs[bsel], idx_sems[bsel]
            ovb, osem = out_bufs[bsel], out_sems[bsel]
            pltpu.make_async_copy(
                idx_hbm.at[pl.ds(b0 * _SEQ, _TOK)], ivb, isem
            ).wait()

            @pl.when(c + 1 < _N_CHUNKS)
            def _():
                b1 = base_b + (c + 1) * _NB
                pltpu.async_copy(
                    idx_hbm.at[pl.ds(b1 * _SEQ, _TOK)],
                    idx_bufs[1 - bsel], idx_sems[1 - bsel],
                )

            @pl.when(c >= 2)
            def _():
                pltpu.make_async_copy(
                    ovb, out_hbm.at[pl.ds(b0, _NB)], osem
                ).wait()

            compute(ivb, ovb)
            pltpu.async_copy(ovb, out_hbm.at[pl.ds(b0, _NB)], osem)
        return 0

    lax.fori_loop(0, _N_CHUNKS // 2, pair_body, 0)

    for bsel in range(2):
        c = _N_CHUNKS - 2 + bsel
        b0 = base_b + c * _NB
        pltpu.make_async_copy(
            out_bufs[bsel], out_hbm.at[pl.ds(b0, _NB)], out_sems[bsel]
        ).wait()


def _sc_gather(table_flat, idx_flat):
    mesh = plsc.VectorSubcoreMesh(core_axis_name="c", subcore_axis_name="s")
    kern = functools.partial(
        pl.kernel,
        mesh=mesh,
        compiler_params=pltpu.CompilerParams(
            needs_layout_passes=False, use_tc_tiling_on_sc=False
        ),
        out_type=jax.ShapeDtypeStruct((_BATCH, _SEQ, _VOCAB), jnp.float32),
        scratch_types=[
            pltpu.VMEM((_VOCAB * _TPAD,), jnp.float32),
            pltpu.VMEM((_TOK,), jnp.int32),
            pltpu.VMEM((_TOK,), jnp.int32),
            pltpu.VMEM((_NB, _SEQ, _VOCAB), jnp.float32),
            pltpu.VMEM((_NB, _SEQ, _VOCAB), jnp.float32),
            pltpu.SemaphoreType.DMA,
            pltpu.SemaphoreType.DMA,
            pltpu.SemaphoreType.DMA,
            pltpu.SemaphoreType.DMA,
            pltpu.SemaphoreType.DMA,
        ],
    )(_gather_body)
    return kern(table_flat, idx_flat)


def kernel(indices, embed_table, W, b):
    table_flat = _fuse_table(embed_table, W, b)
    return _sc_gather(table_flat, indices.reshape(_N))
